# Initial kernel scaffold; baseline (speedup 1.0000x reference)
import jax
import jax.numpy as jnp
from jax.experimental import pallas as pl


def kernel(x, y, edge_index, deep_walk_emb, label_input_mask, gcn_W0, gcn_b0, gcn_W1, gcn_b1, label_Ws, label_bs, fusion_W, fusion_b):
    raise NotImplementedError("write your pallas kernel here")


# speedup vs baseline: 7.9542x; 7.9542x over previous
import functools

import jax
import jax.numpy as jnp
from jax import lax
from jax.experimental import pallas as pl
from jax.experimental.pallas import tpu as pltpu
from jax.experimental.pallas import tpu_sc as plsc

_N = 10000
_E = 320000
_NPAD = 10240
_ROWS_PER_TILE = _NPAD // 16
_NW = 32
_EPW = 10112
_NBLK = _EPW // 128
_EPAD = _NW * _EPW
_F32 = jnp.float32


def _sc_mesh():
    return plsc.VectorSubcoreMesh(core_axis_name="c", subcore_axis_name="s")


def _make_prop(F):
    @functools.partial(
        pl.kernel,
        mesh=_sc_mesh(),
        compiler_params=pltpu.CompilerParams(use_tc_tiling_on_sc=False),
        out_type=jax.ShapeDtypeStruct((2 * _NPAD, F), _F32),
        scratch_types=[
            pltpu.VMEM((128,), jnp.int32),
            pltpu.VMEM((128,), jnp.int32),
            pltpu.VMEM((128, F), _F32),
            pltpu.VMEM_SHARED((_NPAD, F), _F32),
            pltpu.SemaphoreType.DMA,
        ],
    )
    def prop(u_hbm, src_hbm, dst_hbm, out_hbm, src_v, dst_v, rows_v, acc_sh, sem):
        cid = lax.axis_index("c")
        sid = lax.axis_index("s")
        wid = cid * 16 + sid

        nvec = (128 * F) // 16

        def zbody(i, carry):
            r = i // (F // 16)
            c = (i % (F // 16)) * 16
            rows_v[r, pl.ds(c, 16)] = jnp.zeros((16,), _F32)
            return carry

        lax.fori_loop(0, nvec, zbody, 0)
        base_row = sid * _ROWS_PER_TILE
        for k in range(_ROWS_PER_TILE // 128):
            pltpu.sync_copy(rows_v, acc_sh.at[pl.ds(base_row + k * 128, 128)])
        plsc.subcore_barrier()

        ebase = wid * _EPW

        def ebody(j, carry):
            off = ebase + j * 128
            pltpu.sync_copy(src_hbm.at[pl.ds(off, 128)], src_v)
            pltpu.sync_copy(dst_hbm.at[pl.ds(off, 128)], dst_v)
            pltpu.async_copy(u_hbm.at[src_v], rows_v, sem).wait()
            pltpu.sync_copy(rows_v, acc_sh.at[dst_v], add=True)
            return carry

        lax.fori_loop(0, _NBLK, ebody, 0)
        plsc.subcore_barrier()

        out_base = cid * _NPAD + base_row
        for k in range(_ROWS_PER_TILE // 128):
            pltpu.sync_copy(acc_sh.at[pl.ds(base_row + k * 128, 128)],
                            out_hbm.at[pl.ds(out_base + k * 128, 128)])

    return prop


def _make_deg():
    @functools.partial(
        pl.kernel,
        mesh=_sc_mesh(),
        compiler_params=pltpu.CompilerParams(use_tc_tiling_on_sc=False),
        out_type=jax.ShapeDtypeStruct((2 * _NPAD, 16), _F32),
        scratch_types=[
            pltpu.VMEM((128,), jnp.int32),
            pltpu.VMEM((128, 16), _F32),
            pltpu.VMEM_SHARED((_NPAD, 16), _F32),
        ],
    )
    def deg(dst_hbm, out_hbm, dst_v, ones_v, acc_sh):
        cid = lax.axis_index("c")
        sid = lax.axis_index("s")
        wid = cid * 16 + sid

        def fill(i, carry):
            ones_v[i, pl.ds(0, 16)] = jnp.ones((16,), _F32)
            return carry

        lax.fori_loop(0, 128, fill, 0)
        base_row = sid * _ROWS_PER_TILE
        zeros_view = ones_v

        def zfill(i, carry):
            ones_v[i, pl.ds(0, 16)] = jnp.zeros((16,), _F32)
            return carry

        lax.fori_loop(0, 128, zfill, 0)
        for k in range(_ROWS_PER_TILE // 128):
            pltpu.sync_copy(zeros_view, acc_sh.at[pl.ds(base_row + k * 128, 128)])
        lax.fori_loop(0, 128, fill, 0)
        plsc.subcore_barrier()

        ebase = wid * _EPW

        def ebody(j, carry):
            off = ebase + j * 128
            pltpu.sync_copy(dst_hbm.at[pl.ds(off, 128)], dst_v)
            pltpu.sync_copy(ones_v, acc_sh.at[dst_v], add=True)
            return carry

        lax.fori_loop(0, _NBLK, ebody, 0)
        plsc.subcore_barrier()

        out_base = cid * _NPAD + base_row
        for k in range(_ROWS_PER_TILE // 128):
            pltpu.sync_copy(acc_sh.at[pl.ds(base_row + k * 128, 128)],
                            out_hbm.at[pl.ds(out_base + k * 128, 128)])

    return deg


_prop128 = _make_prop(128)
_prop64 = _make_prop(64)
_deg = _make_deg()


def _dot(a, b):
    return jnp.dot(a, b, preferred_element_type=_F32)


def _prep_body(degp_ref, x_ref, y_ref, w0_ref, l0_ref, dinv_ref, ufp_ref, ulp_ref):
    d = degp_ref[0:_NPAD, :] + degp_ref[_NPAD:2 * _NPAD, :] + 1.0
    dinv = lax.rsqrt(d)
    dinv_ref[...] = dinv
    dv = dinv[0:_N, 0:1]
    ufp_ref[...] = dv * _dot(x_ref[...], w0_ref[...])
    ulp_ref[...] = dv * _dot(y_ref[...], l0_ref[...])


def _fpmid_body(accp_ref, u_ref, dinv_ref, b_ref, w_ref, out_ref):
    dv = dinv_ref[0:_N, 0:1]
    s = accp_ref[0:_N, :] + accp_ref[_NPAD:_NPAD + _N, :] + u_ref[...]
    h = dv * s + b_ref[...]
    out_ref[...] = dv * _dot(h, w_ref[...])


def _fpend_body(accp_ref, u_ref, dinv_ref, b_ref, out_ref):
    dv = dinv_ref[0:_N, 0:1]
    s = accp_ref[0:_N, :] + accp_ref[_NPAD:_NPAD + _N, :] + u_ref[...]
    out_ref[...] = dv * s + b_ref[...]


def _lpstep_body(accp_ref, u_ref, dinv_ref, b_ref, y_ref, mf_ref, w_ref, out_ref):
    dv = dinv_ref[0:_N, 0:1]
    s = accp_ref[0:_N, :] + accp_ref[_NPAD:_NPAD + _N, :] + u_ref[...]
    xl = dv * s + b_ref[...]
    mf = mf_ref[...]
    xl = mf * y_ref[...] + (1.0 - mf) * xl
    out_ref[...] = dv * _dot(xl, w_ref[...])


def _final_body(accp_ref, u_ref, dinv_ref, b_ref, y_ref, mf_ref, h2_ref, dw_ref,
                wf_ref, fb_ref, out_ref):
    dv = dinv_ref[0:_N, 0:1]
    s = accp_ref[0:_N, :] + accp_ref[_NPAD:_NPAD + _N, :] + u_ref[...]
    xl = dv * s + b_ref[...]
    mf = mf_ref[...]
    xl = mf * y_ref[...] + (1.0 - mf) * xl
    logits = (_dot(h2_ref[...], wf_ref[0:128, :])
              + _dot(xl, wf_ref[128:192, :])
              + _dot(dw_ref[...], wf_ref[192:256, :])
              + fb_ref[...])
    out_ref[...] = 1.0 / (1.0 + jnp.exp(-logits))


def _tc(body, out_shape, *args):
    return pl.pallas_call(body, out_shape=out_shape)(*args)


def kernel(x, y, edge_index, deep_walk_emb, label_input_mask,
           gcn_W0, gcn_b0, gcn_W1, gcn_b1, label_Ws, label_bs,
           fusion_W, fusion_b):
    src = edge_index[0]
    dst = edge_index[1]
    npad_extra = _NPAD - _N
    pad_e = _EPAD - _E
    srcp = jnp.concatenate([src, jnp.zeros((pad_e,), jnp.int32)])
    dstp = jnp.concatenate(
        [dst, _N + (jnp.arange(pad_e, dtype=jnp.int32) % npad_extra)])
    mf = label_input_mask.astype(_F32)[:, None]

    degp = _deg(dstp)

    dinv16, ufp, ulp = _tc(
        _prep_body,
        (jax.ShapeDtypeStruct((_NPAD, 16), _F32),
         jax.ShapeDtypeStruct((_N, 128), _F32),
         jax.ShapeDtypeStruct((_N, 64), _F32)),
        degp, x, y, gcn_W0, label_Ws[0])

    accp = _prop128(ufp, srcp, dstp)
    ufp2 = _tc(_fpmid_body, jax.ShapeDtypeStruct((_N, 128), _F32),
               accp, ufp, dinv16, gcn_b0[None, :], gcn_W1)
    accp = _prop128(ufp2, srcp, dstp)
    h2 = _tc(_fpend_body, jax.ShapeDtypeStruct((_N, 128), _F32),
             accp, ufp2, dinv16, gcn_b1[None, :])

    u = ulp
    for j in range(9):
        a = _prop64(u, srcp, dstp)
        u = _tc(_lpstep_body, jax.ShapeDtypeStruct((_N, 64), _F32),
                a, u, dinv16, label_bs[j][None, :], y, mf, label_Ws[j + 1])
    a = _prop64(u, srcp, dstp)
    out = _tc(_final_body, jax.ShapeDtypeStruct((_N, 64), _F32),
              a, u, dinv16, label_bs[9][None, :], y, mf, h2,
              deep_walk_emb, fusion_W, fusion_b[None, :])
    return out
